# baseline (speedup 1.0000x reference)
import functools

import jax
import jax.numpy as jnp
from jax import lax
from jax.experimental import pallas as pl
from jax.experimental.pallas import tpu as pltpu
from jax.experimental.pallas import tpu_sc as plsc

N = 10000
E = 320000
D = 128
H = 256

NC = 2
NS = 16
NW = NC * NS
K = 128
CH = 80
CHP = 40
EP = NW * CH * K
NP = 10240
RPT = NP // NS

@functools.lru_cache(maxsize=None)
def _mesh():
    return plsc.VectorSubcoreMesh(
        core_axis_name="c", subcore_axis_name="s", num_cores=NC, num_subcores=NS
    )


def _zero_slice_chunks():
    out = []
    off = 0
    while off < RPT:
        sz = min(K, RPT - off)
        out.append((off, sz))
        off += sz
    return out


def _zero_vmem(ref, rows, cols):
    z = jnp.zeros((16,), jnp.float32)

    def body(i, _):
        r = i // (cols // 16)
        c = (i % (cols // 16)) * 16
        ref[r, pl.ds(c, 16)] = z
        return 0

    lax.fori_loop(0, rows * (cols // 16), body, 0)


@functools.lru_cache(maxsize=None)
def _sc_hist_kernel():
    return pl.kernel(
        _sc_hist_body,
        out_type=jax.ShapeDtypeStruct((NC, NP, 16), jnp.float32),
        mesh=_mesh(),
        scratch_types=[
            pltpu.VMEM((CH, K), jnp.int32),
            pltpu.VMEM((K, 16), jnp.float32),
            pltpu.VMEM_SHARED((NP, 16), jnp.float32),
        ],
    )


def _sc_hist_body(dst_hbm, out_hbm, dst_v, ones_v, deg_sh):
    cid = lax.axis_index("c")
    sid = lax.axis_index("s")
    wid = cid * NS + sid

    _zero_vmem(ones_v, K, 16)
    for off, sz in _zero_slice_chunks():
        pltpu.sync_copy(
            ones_v.at[pl.ds(0, sz)], deg_sh.at[pl.ds(sid * RPT + off, sz)]
        )

    one = jnp.ones((16,), jnp.float32)

    def fill(i, _):
        ones_v[i, pl.ds(0, 16)] = one
        return 0

    lax.fori_loop(0, K, fill, 0)
    plsc.subcore_barrier()

    pltpu.sync_copy(dst_hbm.at[wid], dst_v)

    def body(j, _):
        pltpu.sync_copy(ones_v, deg_sh.at[dst_v.at[j]], add=True)
        return 0

    lax.fori_loop(0, CH, body, 0)
    plsc.subcore_barrier()
    pltpu.sync_copy(
        deg_sh.at[pl.ds(sid * RPT, RPT)], out_hbm.at[cid].at[pl.ds(sid * RPT, RPT)]
    )


@functools.lru_cache(maxsize=None)
def _sc_agg_kernel():
    return pl.kernel(
        _sc_agg_body,
        out_type=jax.ShapeDtypeStruct((NC, NP, D), jnp.float32),
        mesh=_mesh(),
        scratch_types=[
            pltpu.VMEM((CHP, K), jnp.int32),
            pltpu.VMEM((CHP, K), jnp.int32),
            pltpu.VMEM((K, D), jnp.float32),
            pltpu.VMEM((K, D), jnp.float32),
            pltpu.VMEM_SHARED((NP, D), jnp.float32),
            pltpu.SemaphoreType.DMA,
            pltpu.SemaphoreType.DMA,
        ],
    )


def _sc_agg_body(
    x_hbm, src_hbm, dst_hbm, out_hbm, src_v, dst_v, rows_a, rows_b, acc_sh,
    gsem, ssem
):
    cid = lax.axis_index("c")
    sid = lax.axis_index("s")
    wid = cid * NS + sid

    _zero_vmem(rows_a, K, D)
    for off, sz in _zero_slice_chunks():
        pltpu.sync_copy(
            rows_a.at[pl.ds(0, sz)], acc_sh.at[pl.ds(sid * RPT + off, sz)]
        )
    plsc.subcore_barrier()

    PP = CHP // 2
    for ph in range(CH // CHP):
        pltpu.sync_copy(src_hbm.at[wid].at[pl.ds(ph * CHP, CHP)], src_v)
        pltpu.sync_copy(dst_hbm.at[wid].at[pl.ds(ph * CHP, CHP)], dst_v)
        pltpu.async_copy(x_hbm.at[src_v.at[0]], rows_a, gsem)

        def body(p, _):
            j0 = 2 * p
            pltpu.make_async_copy(x_hbm.at[src_v.at[j0]], rows_a, gsem).wait()

            @pl.when(p >= 1)
            def _():
                pltpu.make_async_copy(
                    rows_b, acc_sh.at[dst_v.at[j0 - 1]], ssem
                ).wait()

            pltpu.async_copy(rows_a, acc_sh.at[dst_v.at[j0]], ssem, add=True)
            pltpu.async_copy(x_hbm.at[src_v.at[j0 + 1]], rows_b, gsem)
            pltpu.make_async_copy(
                x_hbm.at[src_v.at[j0 + 1]], rows_b, gsem
            ).wait()
            pltpu.make_async_copy(rows_a, acc_sh.at[dst_v.at[j0]], ssem).wait()
            pltpu.async_copy(rows_b, acc_sh.at[dst_v.at[j0 + 1]], ssem, add=True)

            @pl.when(p + 1 < PP)
            def _():
                pltpu.async_copy(x_hbm.at[src_v.at[j0 + 2]], rows_a, gsem)

            return 0

        lax.fori_loop(0, PP, body, 0)
        pltpu.make_async_copy(rows_b, acc_sh.at[dst_v.at[CHP - 1]], ssem).wait()
    plsc.subcore_barrier()
    pltpu.sync_copy(
        acc_sh.at[pl.ds(sid * RPT, RPT)], out_hbm.at[cid].at[pl.ds(sid * RPT, RPT)]
    )


def _tc_scale_body(degp_ref, emb_ref, dinv_ref, y0_ref):
    deg = degp_ref[0] + degp_ref[1] + 1.0
    dinv = lax.rsqrt(deg)
    dinv_ref[...] = dinv
    y0_ref[...] = emb_ref[...] * dinv


def _tc_scale(degp, emb_p):
    return pl.pallas_call(
        _tc_scale_body,
        out_shape=(
            jax.ShapeDtypeStruct((NP, 1), jnp.float32),
            jax.ShapeDtypeStruct((NP, D), jnp.float32),
        ),
    )(degp, emb_p)


def _tc_mm_body(aggp_ref, y0_ref, dinv_ref, W1_ref, b1_ref, W2_ref, y1_ref):
    dinv = dinv_ref[...]
    g1 = (aggp_ref[0] + aggp_ref[1] + y0_ref[...]) * dinv
    h1 = jnp.dot(g1, W1_ref[...], preferred_element_type=jnp.float32) + b1_ref[...]
    x2 = jnp.dot(h1, W2_ref[...], preferred_element_type=jnp.float32)
    y1_ref[...] = x2 * dinv


def _tc_mm(aggp, y0, dinv, W1, b1, W2):
    blk = 2048
    grid = NP // blk
    return pl.pallas_call(
        _tc_mm_body,
        grid=(grid,),
        in_specs=[
            pl.BlockSpec((2, blk, D), lambda i: (0, i, 0)),
            pl.BlockSpec((blk, D), lambda i: (i, 0)),
            pl.BlockSpec((blk, 1), lambda i: (i, 0)),
            pl.BlockSpec((D, H), lambda i: (0, 0)),
            pl.BlockSpec((1, H), lambda i: (0, 0)),
            pl.BlockSpec((H, D), lambda i: (0, 0)),
        ],
        out_specs=pl.BlockSpec((blk, D), lambda i: (i, 0)),
        out_shape=jax.ShapeDtypeStruct((NP, D), jnp.float32),
    )(aggp, y0, dinv, W1, b1, W2)


def _tc_bn_body(aggp_ref, y1_ref, dinv_ref, b2_ref, gamma_ref, beta_ref, out_ref):
    h2 = (aggp_ref[0] + aggp_ref[1] + y1_ref[...]) * dinv_ref[...] + b2_ref[...]
    mean = jnp.mean(h2, axis=0, keepdims=True)
    var = jnp.mean(h2 * h2, axis=0, keepdims=True) - mean * mean
    inv = lax.rsqrt(var + 1e-5)
    out_ref[...] = (h2 - mean) * inv * gamma_ref[...] + beta_ref[...]


def _tc_bn(aggp, y1, dinv, b2, gamma, beta):
    return pl.pallas_call(
        _tc_bn_body,
        grid=(1,),
        in_specs=[
            pl.BlockSpec((2, N, D), lambda i: (0, 0, 0)),
            pl.BlockSpec((N, D), lambda i: (0, 0)),
            pl.BlockSpec((N, 1), lambda i: (0, 0)),
            pl.BlockSpec((1, D), lambda i: (0, 0)),
            pl.BlockSpec((1, D), lambda i: (0, 0)),
            pl.BlockSpec((1, D), lambda i: (0, 0)),
        ],
        out_specs=pl.BlockSpec((N, D), lambda i: (0, 0)),
        out_shape=jax.ShapeDtypeStruct((N, D), jnp.float32),
    )(aggp, y1, dinv, b2, gamma, beta)


def kernel(edge_index, emb, W1, b1, W2, b2, gamma, beta):
    src = edge_index[0].astype(jnp.int32)
    dst = edge_index[1].astype(jnp.int32)
    src_p = jnp.concatenate([src, jnp.zeros((EP - E,), jnp.int32)]).reshape(NW, CH, K)
    dst_p = jnp.concatenate(
        [dst, jnp.full((EP - E,), N, jnp.int32)]
    ).reshape(NW, CH, K)
    emb_p = jnp.concatenate([emb, jnp.zeros((NP - N, D), emb.dtype)], axis=0)

    degp = _sc_hist_kernel()(dst_p)[:, :, :1]
    dinv, y0 = _tc_scale(degp, emb_p)
    agg0 = _sc_agg_kernel()(y0, src_p, dst_p)
    y1 = _tc_mm(agg0, y0, dinv, W1, b1.reshape(1, H), W2)
    agg1 = _sc_agg_kernel()(y1, src_p, dst_p)
    out = _tc_bn(
        agg1, y1, dinv, b2.reshape(1, D), gamma.reshape(1, D), beta.reshape(1, D)
    )
    return out


# speedup vs baseline: 3.3350x; 3.3350x over previous
import functools

import jax
import jax.numpy as jnp
from jax import lax
from jax.experimental import pallas as pl
from jax.experimental.pallas import tpu as pltpu
from jax.experimental.pallas import tpu_sc as plsc

N = 10000
E = 320000
D = 128
H = 256

NC = 2
NS = 16
NW = NC * NS
K = 128
CH = 80
CHP = 40
EP = NW * CH * K
NP = 10240
RPT = NP // NS

@functools.lru_cache(maxsize=None)
def _mesh():
    return plsc.VectorSubcoreMesh(
        core_axis_name="c", subcore_axis_name="s", num_cores=NC, num_subcores=NS
    )


def _zero_slice_chunks():
    out = []
    off = 0
    while off < RPT:
        sz = min(K, RPT - off)
        out.append((off, sz))
        off += sz
    return out


def _zero_vmem(ref, rows, cols):
    z = jnp.zeros((16,), jnp.float32)

    def body(i, _):
        r = i // (cols // 16)
        c = (i % (cols // 16)) * 16
        ref[r, pl.ds(c, 16)] = z
        return 0

    lax.fori_loop(0, rows * (cols // 16), body, 0)


@functools.lru_cache(maxsize=None)
def _sc_hist_kernel():
    return pl.kernel(
        _sc_hist_body,
        out_type=jax.ShapeDtypeStruct((NC, NP, 16), jnp.float32),
        mesh=_mesh(),
        scratch_types=[
            pltpu.VMEM((CH, K), jnp.int32),
            pltpu.VMEM((K, 16), jnp.float32),
            pltpu.VMEM_SHARED((NP, 16), jnp.float32),
        ],
    )


def _sc_hist_body(dst_hbm, out_hbm, dst_v, ones_v, deg_sh):
    cid = lax.axis_index("c")
    sid = lax.axis_index("s")
    wid = cid * NS + sid

    _zero_vmem(ones_v, K, 16)
    for off, sz in _zero_slice_chunks():
        pltpu.sync_copy(
            ones_v.at[pl.ds(0, sz)], deg_sh.at[pl.ds(sid * RPT + off, sz)]
        )

    one = jnp.ones((16,), jnp.float32)

    def fill(i, _):
        ones_v[i, pl.ds(0, 16)] = one
        return 0

    lax.fori_loop(0, K, fill, 0)
    plsc.subcore_barrier()

    pltpu.sync_copy(dst_hbm.at[wid], dst_v)

    def body(j, _):
        pltpu.sync_copy(ones_v, deg_sh.at[dst_v.at[j]], add=True)
        return 0

    lax.fori_loop(0, CH, body, 0)
    plsc.subcore_barrier()
    pltpu.sync_copy(
        deg_sh.at[pl.ds(sid * RPT, RPT)], out_hbm.at[cid].at[pl.ds(sid * RPT, RPT)]
    )


@functools.lru_cache(maxsize=None)
def _sc_agg_kernel():
    return pl.kernel(
        _sc_agg_body,
        out_type=jax.ShapeDtypeStruct((NC, NP, D), jnp.float32),
        mesh=_mesh(),
        scratch_types=[
            pltpu.VMEM((CHP, K), jnp.int32),
            pltpu.VMEM((CHP, K), jnp.int32),
            pltpu.VMEM((K, D), jnp.float32),
            pltpu.VMEM((K, D), jnp.float32),
            pltpu.VMEM_SHARED((NP, D), jnp.float32),
            pltpu.SemaphoreType.DMA,
            pltpu.SemaphoreType.DMA,
        ],
    )


def _sc_agg_body(
    x_hbm, src_hbm, dst_hbm, out_hbm, src_v, dst_v, rows_a, rows_b, acc_sh,
    gsem, ssem
):
    cid = lax.axis_index("c")
    sid = lax.axis_index("s")
    wid = cid * NS + sid

    _zero_vmem(rows_a, K, D)
    for off, sz in _zero_slice_chunks():
        pltpu.sync_copy(
            rows_a.at[pl.ds(0, sz)], acc_sh.at[pl.ds(sid * RPT + off, sz)]
        )
    plsc.subcore_barrier()

    PP = CHP // 2
    for ph in range(CH // CHP):
        pltpu.sync_copy(src_hbm.at[wid].at[pl.ds(ph * CHP, CHP)], src_v)
        pltpu.sync_copy(dst_hbm.at[wid].at[pl.ds(ph * CHP, CHP)], dst_v)
        pltpu.async_copy(x_hbm.at[src_v.at[0]], rows_a, gsem)

        def body(p, _):
            j0 = 2 * p
            pltpu.make_async_copy(x_hbm.at[src_v.at[j0]], rows_a, gsem).wait()

            @pl.when(p >= 1)
            def _():
                pltpu.make_async_copy(
                    rows_b, acc_sh.at[dst_v.at[j0 - 1]], ssem
                ).wait()

            pltpu.async_copy(rows_a, acc_sh.at[dst_v.at[j0]], ssem, add=True)
            pltpu.async_copy(x_hbm.at[src_v.at[j0 + 1]], rows_b, gsem)
            pltpu.make_async_copy(
                x_hbm.at[src_v.at[j0 + 1]], rows_b, gsem
            ).wait()
            pltpu.make_async_copy(rows_a, acc_sh.at[dst_v.at[j0]], ssem).wait()
            pltpu.async_copy(rows_b, acc_sh.at[dst_v.at[j0 + 1]], ssem, add=True)

            @pl.when(p + 1 < PP)
            def _():
                pltpu.async_copy(x_hbm.at[src_v.at[j0 + 2]], rows_a, gsem)

            return 0

        lax.fori_loop(0, PP, body, 0)
        pltpu.make_async_copy(rows_b, acc_sh.at[dst_v.at[CHP - 1]], ssem).wait()
    plsc.subcore_barrier()
    pltpu.sync_copy(
        acc_sh.at[pl.ds(sid * RPT, RPT)], out_hbm.at[cid].at[pl.ds(sid * RPT, RPT)]
    )


def _tc_scale_body(degp_ref, emb_ref, dinv_ref, y0_ref):
    deg = degp_ref[0] + degp_ref[1] + 1.0
    dinv = lax.rsqrt(deg)
    dinv_ref[...] = dinv
    y0_ref[...] = emb_ref[...] * dinv


def _tc_scale(degp, emb_p):
    return pl.pallas_call(
        _tc_scale_body,
        out_shape=(
            jax.ShapeDtypeStruct((NP, 1), jnp.float32),
            jax.ShapeDtypeStruct((NP, D), jnp.float32),
        ),
    )(degp, emb_p)


def _tc_mm_body(aggp_ref, y0_ref, dinv_ref, W1_ref, b1_ref, W2_ref, y1_ref):
    dinv = dinv_ref[...]
    g1 = (aggp_ref[0] + aggp_ref[1] + y0_ref[...]) * dinv
    h1 = jnp.dot(g1, W1_ref[...], preferred_element_type=jnp.float32) + b1_ref[...]
    x2 = jnp.dot(h1, W2_ref[...], preferred_element_type=jnp.float32)
    y1_ref[...] = x2 * dinv


def _tc_mm(aggp, y0, dinv, W1, b1, W2):
    blk = 2048
    grid = NP // blk
    return pl.pallas_call(
        _tc_mm_body,
        grid=(grid,),
        in_specs=[
            pl.BlockSpec((2, blk, D), lambda i: (0, i, 0)),
            pl.BlockSpec((blk, D), lambda i: (i, 0)),
            pl.BlockSpec((blk, 1), lambda i: (i, 0)),
            pl.BlockSpec((D, H), lambda i: (0, 0)),
            pl.BlockSpec((1, H), lambda i: (0, 0)),
            pl.BlockSpec((H, D), lambda i: (0, 0)),
        ],
        out_specs=pl.BlockSpec((blk, D), lambda i: (i, 0)),
        out_shape=jax.ShapeDtypeStruct((NP, D), jnp.float32),
    )(aggp, y0, dinv, W1, b1, W2)


def _tc_bn_body(aggp_ref, y1_ref, dinv_ref, b2_ref, gamma_ref, beta_ref, out_ref):
    h2 = (aggp_ref[0] + aggp_ref[1] + y1_ref[...]) * dinv_ref[...] + b2_ref[...]
    mean = jnp.mean(h2, axis=0, keepdims=True)
    var = jnp.mean(h2 * h2, axis=0, keepdims=True) - mean * mean
    inv = lax.rsqrt(var + 1e-5)
    out_ref[...] = (h2 - mean) * inv * gamma_ref[...] + beta_ref[...]


def _tc_bn(aggp, y1, dinv, b2, gamma, beta):
    return pl.pallas_call(
        _tc_bn_body,
        grid=(1,),
        in_specs=[
            pl.BlockSpec((2, N, D), lambda i: (0, 0, 0)),
            pl.BlockSpec((N, D), lambda i: (0, 0)),
            pl.BlockSpec((N, 1), lambda i: (0, 0)),
            pl.BlockSpec((1, D), lambda i: (0, 0)),
            pl.BlockSpec((1, D), lambda i: (0, 0)),
            pl.BlockSpec((1, D), lambda i: (0, 0)),
        ],
        out_specs=pl.BlockSpec((N, D), lambda i: (0, 0)),
        out_shape=jax.ShapeDtypeStruct((N, D), jnp.float32),
    )(aggp, y1, dinv, b2, gamma, beta)


def kernel(edge_index, emb, W1, b1, W2, b2, gamma, beta):
    src = edge_index[0].astype(jnp.int32)
    dst = edge_index[1].astype(jnp.int32)
    pad = jnp.arange(EP - E, dtype=jnp.int32)
    src_p = jnp.concatenate([src, pad % N]).reshape(NW, CH, K)
    dst_p = jnp.concatenate([dst, N + pad % (NP - N)]).reshape(NW, CH, K)
    emb_p = jnp.concatenate([emb, jnp.zeros((NP - N, D), emb.dtype)], axis=0)

    degp = _sc_hist_kernel()(dst_p)[:, :, :1]
    dinv, y0 = _tc_scale(degp, emb_p)
    agg0 = _sc_agg_kernel()(y0, src_p, dst_p)
    y1 = _tc_mm(agg0, y0, dinv, W1, b1.reshape(1, H), W2)
    agg1 = _sc_agg_kernel()(y1, src_p, dst_p)
    out = _tc_bn(
        agg1, y1, dinv, b2.reshape(1, D), gamma.reshape(1, D), beta.reshape(1, D)
    )
    return out
